# baseline (speedup 1.0000x reference)
import functools

import numpy as np
import jax
import jax.numpy as jnp
from jax import lax
from jax.experimental import pallas as pl
from jax.experimental.pallas import tpu as pltpu
from jax.experimental.pallas import tpu_sc as plsc

_SUB = 1000
_PAD = 1024
_K = 5


def _rotl32(x, r):
    return (x << np.uint32(r)) | (x >> np.uint32(32 - r))


def _threefry2x32(k0, k1, x0, x1):
    x0 = np.asarray(x0, np.uint32).copy()
    x1 = np.asarray(x1, np.uint32).copy()
    ks = [np.uint32(k0), np.uint32(k1),
          np.uint32(k0) ^ np.uint32(k1) ^ np.uint32(0x1BD11BDA)]
    rotations = [(13, 15, 26, 6), (17, 29, 16, 24)]
    x0 += ks[0]
    x1 += ks[1]
    for i in range(5):
        for r in rotations[i % 2]:
            x0 += x1
            x1 = _rotl32(x1, r)
            x1 ^= x0
        x0 += ks[(i + 1) % 3]
        x1 += ks[(i + 2) % 3] + np.uint32(i + 1)
    return x0, x1


def _fry_bits(keydata, n):
    o0, o1 = _threefry2x32(keydata[0], keydata[1],
                           np.zeros(n, np.uint32),
                           np.arange(n, dtype=np.uint32))
    return o0 ^ o1


def _fry_split(keydata, num):
    o0, o1 = _threefry2x32(keydata[0], keydata[1],
                           np.zeros(num, np.uint32),
                           np.arange(num, dtype=np.uint32))
    return np.stack([o0, o1], axis=1)


_perm_cache = {}


def _subsample_indices(batch_size: int) -> np.ndarray:
    if batch_size not in _perm_cache:
        keydata = np.array([0, 42], dtype=np.uint32)
        num_rounds = int(np.ceil(
            3 * np.log(max(1, batch_size)) / np.log(2**32 - 1)))
        x = np.arange(batch_size, dtype=np.int32)
        for _ in range(num_rounds):
            ks = _fry_split(keydata, 2)
            keydata, subkey = ks[0], ks[1]
            sort_keys = _fry_bits(subkey, batch_size)
            x = x[np.argsort(sort_keys, kind="stable")]
        _perm_cache[batch_size] = x[:_SUB]
    return _perm_cache[batch_size]


_NW = 16
_ROWS_W = _PAD // _NW

_sc_cache = {}


def _sc_kernels(num_colors: int):
    if num_colors in _sc_cache:
        return _sc_cache[num_colors]

    mesh = plsc.VectorSubcoreMesh(core_axis_name="c", subcore_axis_name="s",
                                  num_cores=1)
    rows_w = _ROWS_W
    pts_w = 3 * rows_w

    @functools.partial(
        pl.kernel,
        mesh=mesh,
        out_type=[
            jax.ShapeDtypeStruct((_PAD, num_colors), jnp.float32),
            jax.ShapeDtypeStruct((3 * _PAD,), jnp.float32),
        ],
        scratch_types=[
            pltpu.VMEM((rows_w,), jnp.int32),
            pltpu.VMEM((rows_w, num_colors), jnp.float32),
            pltpu.VMEM((pts_w,), jnp.int32),
            pltpu.VMEM((pts_w,), jnp.float32),
            pltpu.SemaphoreType.DMA,
            pltpu.SemaphoreType.DMA,
        ],
    )
    def gather_k(idx_hbm, table_hbm, pts_hbm, out_rows_hbm, out_pts_hbm,
                 idx_v, rows_v, pidx_v, pvals_v, sem_r, sem_p):
        wid = lax.axis_index("s")
        base = wid * rows_w
        pbase = _PAD + wid * pts_w
        pltpu.sync_copy(idx_hbm.at[pl.ds(base, rows_w)], idx_v)
        cp_r = pltpu.async_copy(table_hbm.at[idx_v], rows_v, sem_r)
        pltpu.sync_copy(idx_hbm.at[pl.ds(pbase, pts_w)], pidx_v)
        cps = [pltpu.async_copy(
                   pts_hbm.at[pidx_v.at[pl.ds(d * rows_w, rows_w)]],
                   pvals_v.at[pl.ds(d * rows_w, rows_w)], sem_p)
               for d in range(3)]
        cp_r.wait()
        pltpu.sync_copy(rows_v, out_rows_hbm.at[pl.ds(base, rows_w)])
        for d in range(3):
            cps[d].wait()
            pltpu.sync_copy(pvals_v.at[pl.ds(d * rows_w, rows_w)],
                            out_pts_hbm.at[pl.ds(d * _PAD + base - _PAD, rows_w)])

    _sc_cache[num_colors] = gather_k
    return _sc_cache[num_colors]


def _select_body(pc_ref, pr_ref, idx_ref):
    D = jnp.zeros((_PAD, _PAD), jnp.float32)
    for d in range(3):
        diff = pc_ref[:, d:d + 1] - pr_ref[d:d + 1, :]
        D = D + diff * diff

    coli = lax.broadcasted_iota(jnp.int32, (_PAD, _PAD), 1)
    valid_row = lax.broadcasted_iota(jnp.int32, (_PAD, 1), 0) < _SUB

    big = jnp.float32(3e38)
    K = lax.bitcast_convert_type(
        (lax.bitcast_convert_type(D, jnp.int32) & jnp.int32(-1024)) | coli,
        jnp.float32)
    K = jnp.where(coli < _SUB, K, big)
    for t in range(_K + 1):
        kmin = jnp.min(K, axis=1, keepdims=True)
        if t > 0:
            idx_t = lax.bitcast_convert_type(kmin, jnp.int32) & jnp.int32(1023)
            idx_ref[:, t - 1:t] = jnp.where(valid_row, idx_t, jnp.int32(-1))
        if t < _K:
            K = jnp.where(K == kmin, big, K)
    idx_ref[:, _K:] = jnp.broadcast_to(jnp.int32(-1), (_PAD, 8 - _K))


_select = pl.pallas_call(
    _select_body,
    out_shape=jax.ShapeDtypeStruct((_PAD, 8), jnp.int32),
)


def _reduce_body(o_ref, idx_ref, out_ref):
    O = o_ref[...]
    G = lax.dot_general(O, O, (((1,), (1,)), ((), ())),
                        preferred_element_type=jnp.float32)
    OO = O * O
    n_col = jnp.sum(OO, axis=1, keepdims=True)
    n_row = lax.dot_general(jnp.ones((1, O.shape[1]), jnp.float32), OO,
                            (((1,), (1,)), ((), ())),
                            preferred_element_type=jnp.float32)
    SQ = n_col + n_row - 2.0 * G

    coli = lax.broadcasted_iota(jnp.int32, (_PAD, _PAD), 1)
    sel = coli == idx_ref[:, 0:1]
    for t in range(1, _K):
        sel = sel | (coli == idx_ref[:, t:t + 1])
    contrib = jnp.where(sel, jnp.sqrt(jnp.maximum(SQ, 0.0)), 0.0)
    out_ref[...] = jnp.broadcast_to(jnp.sum(contrib), (1, 1))


_reduce = pl.pallas_call(
    _reduce_body,
    out_shape=jax.ShapeDtypeStruct((1, 1), jnp.float32),
)


def kernel(outputs, points, k):
    batch, num_colors = outputs.shape
    perm = _subsample_indices(batch)
    perm_pad = np.concatenate(
        [perm, np.full((_PAD - _SUB,), perm[0], np.int32)])
    tiles = perm_pad.reshape(_NW, 1, _ROWS_W)
    flat_idx = (np.arange(3, dtype=np.int32).reshape(1, 3, 1) * batch
                + tiles).reshape(-1).copy()

    gather_k = _sc_kernels(num_colors)
    all_idx = np.concatenate([perm_pad, flat_idx])
    rows, pts_flat = gather_k(jnp.asarray(all_idx), outputs,
                              points.T.reshape(-1))

    pts_t = pts_flat.reshape(3, _PAD)
    pr = jnp.pad(pts_t, ((0, 5), (0, 0)))
    pc = jnp.pad(pts_t.T, ((0, 0), (0, 5)))

    nbr_idx = _select(pc, pr)
    total = _reduce(rows, nbr_idx)[0, 0]
    loss = total / jnp.float32(_SUB * _K)
    return loss * (jnp.asarray(k, loss.dtype) / _K)


# speedup vs baseline: 1.0624x; 1.0624x over previous
import functools

import numpy as np
import jax
import jax.numpy as jnp
from jax import lax
from jax.experimental import pallas as pl
from jax.experimental.pallas import tpu as pltpu
from jax.experimental.pallas import tpu_sc as plsc

_SUB = 1000
_PAD = 1024
_K = 5


def _rotl32(x, r):
    return (x << np.uint32(r)) | (x >> np.uint32(32 - r))


def _threefry2x32(k0, k1, x0, x1):
    x0 = np.asarray(x0, np.uint32).copy()
    x1 = np.asarray(x1, np.uint32).copy()
    ks = [np.uint32(k0), np.uint32(k1),
          np.uint32(k0) ^ np.uint32(k1) ^ np.uint32(0x1BD11BDA)]
    rotations = [(13, 15, 26, 6), (17, 29, 16, 24)]
    x0 += ks[0]
    x1 += ks[1]
    for i in range(5):
        for r in rotations[i % 2]:
            x0 += x1
            x1 = _rotl32(x1, r)
            x1 ^= x0
        x0 += ks[(i + 1) % 3]
        x1 += ks[(i + 2) % 3] + np.uint32(i + 1)
    return x0, x1


def _fry_bits(keydata, n):
    o0, o1 = _threefry2x32(keydata[0], keydata[1],
                           np.zeros(n, np.uint32),
                           np.arange(n, dtype=np.uint32))
    return o0 ^ o1


def _fry_split(keydata, num):
    o0, o1 = _threefry2x32(keydata[0], keydata[1],
                           np.zeros(num, np.uint32),
                           np.arange(num, dtype=np.uint32))
    return np.stack([o0, o1], axis=1)


_perm_cache = {}


def _subsample_indices(batch_size: int) -> np.ndarray:
    if batch_size not in _perm_cache:
        keydata = np.array([0, 42], dtype=np.uint32)
        num_rounds = int(np.ceil(
            3 * np.log(max(1, batch_size)) / np.log(2**32 - 1)))
        x = np.arange(batch_size, dtype=np.int32)
        for _ in range(num_rounds):
            ks = _fry_split(keydata, 2)
            keydata, subkey = ks[0], ks[1]
            sort_keys = _fry_bits(subkey, batch_size)
            x = x[np.argsort(sort_keys, kind="stable")]
        _perm_cache[batch_size] = x[:_SUB]
    return _perm_cache[batch_size]


_NW = 16
_ROWS_W = _PAD // _NW

_sc_cache = {}


def _sc_kernels(num_colors: int):
    if num_colors in _sc_cache:
        return _sc_cache[num_colors]

    mesh = plsc.VectorSubcoreMesh(core_axis_name="c", subcore_axis_name="s",
                                  num_cores=1)
    rows_w = _ROWS_W
    pts_w = 3 * rows_w

    @functools.partial(
        pl.kernel,
        mesh=mesh,
        out_type=jax.ShapeDtypeStruct((_PAD, num_colors), jnp.float32),
        scratch_types=[
            pltpu.VMEM((rows_w,), jnp.int32),
            pltpu.VMEM((rows_w, num_colors), jnp.float32),
            pltpu.SemaphoreType.DMA,
        ],
    )
    def gather_rows(row_idx_hbm, table_hbm, out_rows_hbm, idx_v, rows_v, sem):
        wid = lax.axis_index("s")
        base = wid * rows_w
        pltpu.sync_copy(row_idx_hbm.at[pl.ds(base, rows_w)], idx_v)
        pltpu.async_copy(table_hbm.at[idx_v], rows_v, sem).wait()
        pltpu.sync_copy(rows_v, out_rows_hbm.at[pl.ds(base, rows_w)])

    @functools.partial(
        pl.kernel,
        mesh=mesh,
        out_type=jax.ShapeDtypeStruct((3 * _PAD,), jnp.float32),
        scratch_types=[
            pltpu.VMEM((pts_w,), jnp.int32),
            pltpu.VMEM((pts_w,), jnp.float32),
            pltpu.SemaphoreType.DMA,
        ],
    )
    def gather_pts(flat_idx_hbm, pts_hbm, out_pts_hbm, pidx_v, pvals_v, sem):
        wid = lax.axis_index("s")
        base = wid * rows_w
        pbase = wid * pts_w
        pltpu.sync_copy(flat_idx_hbm.at[pl.ds(pbase, pts_w)], pidx_v)
        cps = [pltpu.async_copy(
                   pts_hbm.at[pidx_v.at[pl.ds(d * rows_w, rows_w)]],
                   pvals_v.at[pl.ds(d * rows_w, rows_w)], sem)
               for d in range(3)]
        for d in range(3):
            cps[d].wait()
            pltpu.sync_copy(pvals_v.at[pl.ds(d * rows_w, rows_w)],
                            out_pts_hbm.at[pl.ds(d * _PAD + base, rows_w)])

    _sc_cache[num_colors] = (gather_rows, gather_pts)
    return _sc_cache[num_colors]


def _select_body(pc_ref, pr_ref, idx_ref):
    D = jnp.zeros((_PAD, _PAD), jnp.float32)
    for d in range(3):
        diff = pc_ref[:, d:d + 1] - pr_ref[d:d + 1, :]
        D = D + diff * diff

    coli = lax.broadcasted_iota(jnp.int32, (_PAD, _PAD), 1)
    valid_row = lax.broadcasted_iota(jnp.int32, (_PAD, 1), 0) < _SUB

    big = jnp.float32(3e38)
    K = lax.bitcast_convert_type(
        (lax.bitcast_convert_type(D, jnp.int32) & jnp.int32(-1024)) | coli,
        jnp.float32)
    K = jnp.where(coli < _SUB, K, big)
    for t in range(_K + 1):
        kmin = jnp.min(K, axis=1, keepdims=True)
        if t > 0:
            idx_t = lax.bitcast_convert_type(kmin, jnp.int32) & jnp.int32(1023)
            idx_ref[:, t - 1:t] = jnp.where(valid_row, idx_t, jnp.int32(-1))
        if t < _K:
            K = jnp.where(K == kmin, big, K)
    idx_ref[:, _K:] = jnp.broadcast_to(jnp.int32(-1), (_PAD, 8 - _K))


_select = pl.pallas_call(
    _select_body,
    out_shape=jax.ShapeDtypeStruct((_PAD, 8), jnp.int32),
)


def _reduce_body(o_ref, idx_ref, out_ref):
    O = o_ref[...]
    G = lax.dot_general(O, O, (((1,), (1,)), ((), ())),
                        preferred_element_type=jnp.float32)
    OO = O * O
    n_col = jnp.sum(OO, axis=1, keepdims=True)
    n_row = lax.dot_general(jnp.ones((1, O.shape[1]), jnp.float32), OO,
                            (((1,), (1,)), ((), ())),
                            preferred_element_type=jnp.float32)
    SQ = n_col + n_row - 2.0 * G

    coli = lax.broadcasted_iota(jnp.int32, (_PAD, _PAD), 1)
    sel = coli == idx_ref[:, 0:1]
    for t in range(1, _K):
        sel = sel | (coli == idx_ref[:, t:t + 1])
    contrib = jnp.where(sel, jnp.sqrt(jnp.maximum(SQ, 0.0)), 0.0)
    out_ref[...] = jnp.broadcast_to(jnp.sum(contrib), (1, 1))


_reduce = pl.pallas_call(
    _reduce_body,
    out_shape=jax.ShapeDtypeStruct((1, 1), jnp.float32),
)


def kernel(outputs, points, k):
    batch, num_colors = outputs.shape
    perm = _subsample_indices(batch)
    perm_pad = np.concatenate(
        [perm, np.full((_PAD - _SUB,), perm[0], np.int32)])
    tiles = perm_pad.reshape(_NW, 1, _ROWS_W)
    flat_idx = (np.arange(3, dtype=np.int32).reshape(1, 3, 1) * batch
                + tiles).reshape(-1).copy()

    gather_rows, gather_pts = _sc_kernels(num_colors)
    pts_flat = gather_pts(jnp.asarray(flat_idx), points.T.reshape(-1))
    rows = gather_rows(jnp.asarray(perm_pad), outputs)

    pts_t = pts_flat.reshape(3, _PAD)
    pr = jnp.pad(pts_t, ((0, 5), (0, 0)))
    pc = jnp.pad(pts_t.T, ((0, 0), (0, 5)))

    nbr_idx = _select(pc, pr)
    total = _reduce(rows, nbr_idx)[0, 0]
    loss = total / jnp.float32(_SUB * _K)
    return loss * (jnp.asarray(k, loss.dtype) / _K)
